# Initial kernel scaffold; baseline (speedup 1.0000x reference)
import jax
import jax.numpy as jnp
from jax.experimental import pallas as pl


def kernel(atoms, pos, edge_index, batch, emb, fc1_w0, fc1_b0, fc2_w0, fc2_b0, fc1_w1, fc1_b1, fc2_w1, fc2_b1, pred_w1, pred_b1, pred_w2, pred_b2):
    raise NotImplementedError("write your pallas kernel here")


# speedup vs baseline: 1.1742x; 1.1742x over previous
import functools

import jax
import jax.numpy as jnp
import numpy as np
from jax import lax
from jax.experimental import pallas as pl
from jax.experimental.pallas import tpu as pltpu
from jax.experimental.pallas import tpu_sc as plsc

N_NODES = 10000
N_EDGES = 160000
EMB = 16
NUM_BESSEL = 8
MLP_DIM = 256
R_MAX = 10.0
P_CUT = 5.0
N_GRAPHS = 64
SQ3 = float(np.sqrt(3.0))
PREF = float(np.sqrt(2.0 / R_MAX))

NC = 2
NS = 16
NW = NC * NS


def _sc_gather(table, idx, chunk):
    B = idx.shape[0]
    D = table.shape[1]
    n_per_w = B // NW
    iters = n_per_w // chunk
    assert n_per_w * NW == B and iters * chunk == n_per_w

    mesh = plsc.VectorSubcoreMesh(core_axis_name="c", subcore_axis_name="s")

    @functools.partial(
        pl.kernel, mesh=mesh,
        out_type=jax.ShapeDtypeStruct((B, D), jnp.float32),
        compiler_params=pltpu.CompilerParams(use_tc_tiling_on_sc=False),
        scratch_types=[
            pltpu.VMEM((chunk,), jnp.int32),
            pltpu.VMEM((chunk, D), jnp.float32),
            pltpu.SemaphoreType.DMA,
        ],
    )
    def k(table_hbm, idx_hbm, out_hbm, idx_v, rows_v, sem):
        wid = lax.axis_index("s") * NC + lax.axis_index("c")
        base = wid * n_per_w

        def body(i, carry):
            off = base + i * chunk
            pltpu.sync_copy(idx_hbm.at[pl.ds(off, chunk)], idx_v)
            pltpu.async_copy(table_hbm.at[idx_v], rows_v, sem).wait()
            pltpu.sync_copy(rows_v, out_hbm.at[pl.ds(off, chunk)])
            return carry

        lax.fori_loop(0, iters, body, 0)

    return k(table, idx)


def _sc_scatter_add(msg, src, zeros, chunk):
    E, D = msg.shape
    N = zeros.shape[0]
    n_per_w = E // NW
    iters = n_per_w // chunk
    rows_per_s = N // NS
    assert n_per_w * NW == E and iters * chunk == n_per_w and rows_per_s * NS == N

    mesh = plsc.VectorSubcoreMesh(core_axis_name="c", subcore_axis_name="s")

    @functools.partial(
        pl.kernel, mesh=mesh,
        out_type=jax.ShapeDtypeStruct((NC, N, D), jnp.float32),
        compiler_params=pltpu.CompilerParams(use_tc_tiling_on_sc=False),
        scratch_types=[
            pltpu.VMEM((chunk,), jnp.int32),
            pltpu.VMEM((chunk, D), jnp.float32),
            pltpu.VMEM_SHARED((N, D), jnp.float32),
        ],
    )
    def k(msg_hbm, src_hbm, zero_hbm, out_hbm, idx_v, rows_v, acc):
        c = lax.axis_index("c")
        s = lax.axis_index("s")
        wid = s * NC + c
        base = wid * n_per_w

        pltpu.sync_copy(zero_hbm.at[pl.ds(s * rows_per_s, rows_per_s)],
                        acc.at[pl.ds(s * rows_per_s, rows_per_s)])
        plsc.subcore_barrier()

        def body(i, carry):
            off = base + i * chunk
            pltpu.sync_copy(src_hbm.at[pl.ds(off, chunk)], idx_v)
            pltpu.sync_copy(msg_hbm.at[pl.ds(off, chunk)], rows_v)
            pltpu.sync_copy(rows_v, acc.at[idx_v], add=True)
            return carry

        lax.fori_loop(0, iters, body, 0)
        plsc.subcore_barrier()

        pltpu.sync_copy(acc.at[pl.ds(s * rows_per_s, rows_per_s)],
                        out_hbm.at[c, pl.ds(s * rows_per_s, rows_per_s)])

    return k(msg, src, zeros)


def _routing(dtype=jnp.float32):
    lE = lax.broadcasted_iota(jnp.int32, (EMB, EMB * EMB), 1)
    rE = lax.broadcasted_iota(jnp.int32, (EMB, EMB * EMB), 0)
    R = (lE // EMB == rE).astype(dtype)
    lS = lax.broadcasted_iota(jnp.int32, (EMB * EMB, EMB), 0)
    rS = lax.broadcasted_iota(jnp.int32, (EMB * EMB, EMB), 1)
    S = (lS % EMB == rS).astype(dtype)
    l2 = lax.broadcasted_iota(jnp.int32, (EMB, 3 * EMB), 1)
    r2 = lax.broadcasted_iota(jnp.int32, (EMB, 3 * EMB), 0)
    R2 = (l2 // 3 == r2).astype(dtype)
    P3n = (((l2 % 3) + 1) % 3 == r2).astype(dtype)
    lZ = lax.broadcasted_iota(jnp.int32, (3 * EMB, EMB), 0)
    rZ = lax.broadcasted_iota(jnp.int32, (3 * EMB, EMB), 1)
    S2 = (lZ // 3 == rZ).astype(dtype)
    return R, S, R2, P3n, S2


def _mm(a, b):
    return jnp.dot(a, b, preferred_element_type=jnp.float32)


def _tp16(x16, wblk, R, S):
    return _mm(_mm(x16, R) * wblk, S)


def _edge_body(first, ps_ref, td_ref, f1_ref, b1_ref, f2_ref, b2_ref, out_ref):
    R, S, R2, P3n, S2 = _routing()
    ps = ps_ref[...]
    blk = td_ref[...]
    pd = blk[:, :16]
    hd = blk[:, 16:]

    vec = ps - pd
    ln = jnp.sqrt(jnp.sum(vec * vec, axis=1, keepdims=True))
    u16 = vec / ln
    x = ln * (1.0 / R_MAX)

    li = lax.broadcasted_iota(jnp.int32, (1, 16), 1)
    nv = jnp.where(li < NUM_BESSEL, (li + 1).astype(jnp.float32) * np.pi, 0.0)
    x2 = x * x
    x4 = x2 * x2
    x5 = x4 * x
    env = 1.0 - 21.0 * x5 + 35.0 * x5 * x - 15.0 * x5 * x2
    env = jnp.where(x < 1.0, env, 0.0)
    ef16 = (PREF * env / ln) * jnp.sin(nv * x)

    hid = jnp.maximum(_mm(ef16, f1_ref[...]) + b1_ref[...], 0.0)
    w = _mm(hid, f2_ref[...]) + b2_ref[...]

    dtile = _mm(u16, P3n)
    y1t = SQ3 * dtile

    if first:
        a0 = 1.0 / np.sqrt(EMB)
        a1 = np.sqrt(3.0) / np.sqrt(EMB)
        h0 = hd
        s = a0 * _tp16(h0, w[:, 0:256], R, S)
        g = a0 * _tp16(h0, w[:, 256:512], R, S)
        t = _tp16(h0, w[:, 512:768], R, S)
        v48 = (a1 / SQ3) * (_mm(t, R2) * y1t)
    else:
        a0 = 1.0 / np.sqrt(2 * EMB)
        a1 = np.sqrt(3.0) / np.sqrt(2 * EMB)
        s_in = hd[:, :16]
        v_in = hd[:, 16:64]
        dot = _mm(v_in * dtile, S2)
        s = a0 * (_tp16(s_in, w[:, 0:256], R, S) + _tp16(dot, w[:, 256:512], R, S))
        g = a0 * (_tp16(s_in, w[:, 512:768], R, S) + _tp16(dot, w[:, 768:1024], R, S))
        t4 = _tp16(s_in, w[:, 1024:1280], R, S)
        v1 = _mm(t4, R2) * y1t
        lk = lax.broadcasted_iota(jnp.int32, (3 * EMB, EMB), 0)
        l2 = lax.broadcasted_iota(jnp.int32, (EMB, 3 * EMB), 1)
        v2 = jnp.zeros_like(v1)
        for kk in range(3):
            Skk = S2 * (lk % 3 == kk).astype(jnp.float32)
            Rkk = R2 * (l2 % 3 == kk).astype(jnp.float32)
            v2 = v2 + _mm(_tp16(_mm(v_in, Skk), w[:, 1280:1536], R, S), Rkk)
        v48 = (a1 / SQ3) * (v1 + v2)

    out_ref[...] = jnp.concatenate([s, g, v48], axis=1)


def _edge_call(first, possrc, tbl_dst, f1, b1, f2, b2):
    T = 256
    E = possrc.shape[0]
    Dt = tbl_dst.shape[1]
    wn = f2.shape[1]
    return pl.pallas_call(
        functools.partial(_edge_body, first),
        grid=(E // T,),
        in_specs=[
            pl.BlockSpec((T, 16), lambda i: (i, 0)),
            pl.BlockSpec((T, Dt), lambda i: (i, 0)),
            pl.BlockSpec((16, MLP_DIM), lambda i: (0, 0)),
            pl.BlockSpec((1, MLP_DIM), lambda i: (0, 0)),
            pl.BlockSpec((MLP_DIM, wn), lambda i: (0, 0)),
            pl.BlockSpec((1, wn), lambda i: (0, 0)),
        ],
        out_specs=pl.BlockSpec((T, 5 * EMB), lambda i: (i, 0)),
        out_shape=jax.ShapeDtypeStruct((E, 5 * EMB), jnp.float32),
    )(possrc, tbl_dst, f1, b1, f2, b2)


def _node_body(a_ref, b_ref, res_ref, out_ref):
    agg = a_ref[...] + b_ref[...]
    sc = agg[:, :EMB]
    s = sc * jax.nn.sigmoid(sc)
    gs = jax.nn.sigmoid(agg[:, EMB:2 * EMB])
    l2 = lax.broadcasted_iota(jnp.int32, (EMB, 3 * EMB), 1)
    r2 = lax.broadcasted_iota(jnp.int32, (EMB, 3 * EMB), 0)
    Rg = (l2 // 3 == r2).astype(jnp.float32)
    v48 = agg[:, 2 * EMB:] * _mm(gs, Rg)
    out_ref[...] = jnp.concatenate([s, v48], axis=1) + res_ref[...]


def _node_call(agg_a, agg_b, res):
    T = 400
    N = agg_a.shape[0]
    return pl.pallas_call(
        _node_body,
        grid=(N // T,),
        in_specs=[
            pl.BlockSpec((T, 5 * EMB), lambda i: (i, 0)),
            pl.BlockSpec((T, 5 * EMB), lambda i: (i, 0)),
            pl.BlockSpec((T, 4 * EMB), lambda i: (i, 0)),
        ],
        out_specs=pl.BlockSpec((T, 4 * EMB), lambda i: (i, 0)),
        out_shape=jax.ShapeDtypeStruct((N, 4 * EMB), jnp.float32),
    )(agg_a, agg_b, res)


def _readout_body(h_ref, batch_ref, w1_ref, b1_ref, w2_ref, b2_ref, out_ref):
    b = batch_ref[...]
    g = lax.broadcasted_iota(jnp.int32, (N_GRAPHS, 1), 0)
    onehot = (b == g).astype(jnp.float32)
    sums = _mm(onehot, h_ref[...][:, :EMB])
    hid = jnp.maximum(_mm(sums, w1_ref[...]) + b1_ref[...], 0.0)
    out_ref[...] = _mm(hid, w2_ref[...]) + b2_ref[...]


def _readout_call(h, batch2d, w1, b1, w2, b2):
    N = h.shape[0]
    return pl.pallas_call(
        _readout_body,
        in_specs=[
            pl.BlockSpec((N, 4 * EMB), lambda: (0, 0)),
            pl.BlockSpec((1, N), lambda: (0, 0)),
            pl.BlockSpec((EMB, EMB), lambda: (0, 0)),
            pl.BlockSpec((1, EMB), lambda: (0, 0)),
            pl.BlockSpec((EMB, 1), lambda: (0, 0)),
            pl.BlockSpec((1, 1), lambda: (0, 0)),
        ],
        out_specs=pl.BlockSpec((N_GRAPHS, 1), lambda: (0, 0)),
        out_shape=jax.ShapeDtypeStruct((N_GRAPHS, 1), jnp.float32),
    )(h, batch2d, w1, b1, w2, b2)


def kernel(atoms, pos, edge_index, batch, emb, fc1_w0, fc1_b0, fc2_w0, fc2_b0,
           fc1_w1, fc1_b1, fc2_w1, fc2_b1, pred_w1, pred_b1, pred_w2, pred_b2):
    f32 = jnp.float32
    src = edge_index[0].astype(jnp.int32)
    dst = edge_index[1].astype(jnp.int32)
    atoms_i = atoms.astype(jnp.int32)

    pos16 = jnp.pad(pos.astype(f32), ((0, 0), (0, 13)))
    zeros80 = jnp.zeros((N_NODES, 5 * EMB), f32)

    BN = 10240
    atoms_p = jnp.pad(atoms_i, (0, BN - N_NODES))

    h0 = _sc_gather(emb.astype(f32), atoms_p, 64)[:N_NODES]
    possrc = _sc_gather(pos16, src, 40)

    f1_0 = jnp.pad(fc1_w0.astype(f32), ((0, 8), (0, 0)))
    f1_1 = jnp.pad(fc1_w1.astype(f32), ((0, 8), (0, 0)))
    b1_0 = fc1_b0.reshape(1, -1).astype(f32)
    b1_1 = fc1_b1.reshape(1, -1).astype(f32)
    b2_0 = fc2_b0.reshape(1, -1).astype(f32)
    b2_1 = fc2_b1.reshape(1, -1).astype(f32)

    tbl0 = jnp.concatenate([pos16, h0], axis=1)
    td0 = _sc_gather(tbl0, dst, 40)
    msg0 = _edge_call(True, possrc, td0, f1_0, b1_0, fc2_w0.astype(f32), b2_0)
    agg0 = _sc_scatter_add(msg0, src, zeros80, 40)
    res0 = jnp.pad(h0, ((0, 0), (0, 3 * EMB)))
    h1 = _node_call(agg0[0], agg0[1], res0)

    tbl1 = jnp.concatenate([pos16, h1], axis=1)
    td1 = _sc_gather(tbl1, dst, 40)
    msg1 = _edge_call(False, possrc, td1, f1_1, b1_1, fc2_w1.astype(f32), b2_1)
    agg1 = _sc_scatter_add(msg1, src, zeros80, 40)
    h2 = _node_call(agg1[0], agg1[1], h1)

    batch2d = batch.astype(jnp.int32).reshape(1, N_NODES)
    return _readout_call(h2, batch2d,
                         pred_w1.astype(f32), pred_b1.reshape(1, -1).astype(f32),
                         pred_w2.astype(f32), pred_b2.reshape(1, -1).astype(f32))


# speedup vs baseline: 1.2412x; 1.0571x over previous
import functools

import jax
import jax.numpy as jnp
import numpy as np
from jax import lax
from jax.experimental import pallas as pl
from jax.experimental.pallas import tpu as pltpu
from jax.experimental.pallas import tpu_sc as plsc

N_NODES = 10000
N_EDGES = 160000
EMB = 16
NUM_BESSEL = 8
MLP_DIM = 256
R_MAX = 10.0
P_CUT = 5.0
N_GRAPHS = 64
SQ3 = float(np.sqrt(3.0))
PREF = float(np.sqrt(2.0 / R_MAX))

NC = 2
NS = 16
NW = NC * NS


def _sc_gather(table, idx3d):
    _, iters, chunk = idx3d.shape
    D = table.shape[1]
    n_per_w = iters * chunk
    B = n_per_w * NW

    mesh = plsc.VectorSubcoreMesh(core_axis_name="c", subcore_axis_name="s")

    @functools.partial(
        pl.kernel, mesh=mesh,
        out_type=jax.ShapeDtypeStruct((B, D), jnp.float32),
        compiler_params=pltpu.CompilerParams(use_tc_tiling_on_sc=False),
        scratch_types=[
            pltpu.VMEM((iters, chunk), jnp.int32),
            pltpu.VMEM((chunk, D), jnp.float32),
            pltpu.SemaphoreType.DMA,
        ],
    )
    def k(table_hbm, idx_hbm, out_hbm, idx_v, rows_v, sem):
        wid = lax.axis_index("s") * NC + lax.axis_index("c")
        base = wid * n_per_w
        pltpu.sync_copy(idx_hbm.at[wid], idx_v)

        def body(i, carry):
            pltpu.async_copy(table_hbm.at[idx_v.at[i]], rows_v, sem).wait()
            pltpu.sync_copy(rows_v, out_hbm.at[pl.ds(base + i * chunk, chunk)])
            return carry

        lax.fori_loop(0, iters, body, 0)

    return k(table, idx3d)


def _sc_scatter_add(msg, src3d, zeros):
    E, D = msg.shape
    N = zeros.shape[0]
    _, iters, chunk = src3d.shape
    n_per_w = iters * chunk
    rows_per_s = N // NS
    assert n_per_w * NW == E and rows_per_s * NS == N

    mesh = plsc.VectorSubcoreMesh(core_axis_name="c", subcore_axis_name="s")

    @functools.partial(
        pl.kernel, mesh=mesh,
        out_type=jax.ShapeDtypeStruct((NC, N, D), jnp.float32),
        compiler_params=pltpu.CompilerParams(use_tc_tiling_on_sc=False),
        scratch_types=[
            pltpu.VMEM((iters, chunk), jnp.int32),
            pltpu.VMEM((chunk, D), jnp.float32),
            pltpu.VMEM_SHARED((N, D), jnp.float32),
        ],
    )
    def k(msg_hbm, src_hbm, zero_hbm, out_hbm, idx_v, rows_v, acc):
        c = lax.axis_index("c")
        s = lax.axis_index("s")
        wid = s * NC + c
        base = wid * n_per_w
        pltpu.sync_copy(src_hbm.at[wid], idx_v)

        pltpu.sync_copy(zero_hbm.at[pl.ds(s * rows_per_s, rows_per_s)],
                        acc.at[pl.ds(s * rows_per_s, rows_per_s)])
        plsc.subcore_barrier()

        def body(i, carry):
            pltpu.sync_copy(msg_hbm.at[pl.ds(base + i * chunk, chunk)], rows_v)
            pltpu.sync_copy(rows_v, acc.at[idx_v.at[i]], add=True)
            return carry

        lax.fori_loop(0, iters, body, 0)
        plsc.subcore_barrier()

        pltpu.sync_copy(acc.at[pl.ds(s * rows_per_s, rows_per_s)],
                        out_hbm.at[c, pl.ds(s * rows_per_s, rows_per_s)])

    return k(msg, src3d, zeros)


def _routing(dtype=jnp.float32):
    lE = lax.broadcasted_iota(jnp.int32, (EMB, EMB * EMB), 1)
    rE = lax.broadcasted_iota(jnp.int32, (EMB, EMB * EMB), 0)
    R = (lE // EMB == rE).astype(dtype)
    lS = lax.broadcasted_iota(jnp.int32, (EMB * EMB, EMB), 0)
    rS = lax.broadcasted_iota(jnp.int32, (EMB * EMB, EMB), 1)
    S = (lS % EMB == rS).astype(dtype)
    l2 = lax.broadcasted_iota(jnp.int32, (EMB, 3 * EMB), 1)
    r2 = lax.broadcasted_iota(jnp.int32, (EMB, 3 * EMB), 0)
    R2 = (l2 // 3 == r2).astype(dtype)
    P3n = (((l2 % 3) + 1) % 3 == r2).astype(dtype)
    lZ = lax.broadcasted_iota(jnp.int32, (3 * EMB, EMB), 0)
    rZ = lax.broadcasted_iota(jnp.int32, (3 * EMB, EMB), 1)
    S2 = (lZ // 3 == rZ).astype(dtype)
    return R, S, R2, P3n, S2


def _mm(a, b):
    return jnp.dot(a, b, preferred_element_type=jnp.float32)


def _tp16(x16, wblk, R, S):
    return _mm(_mm(x16, R) * wblk, S)


def _edge_body(first, ps_ref, td_ref, f1_ref, b1_ref, f2_ref, b2_ref, out_ref):
    R, S, R2, P3n, S2 = _routing()
    ps = ps_ref[...]
    blk = td_ref[...]
    pd = blk[:, :16]
    hd = blk[:, 16:]

    vec = ps - pd
    ln = jnp.sqrt(jnp.sum(vec * vec, axis=1, keepdims=True))
    ln = jnp.maximum(ln, 1e-20)
    u16 = vec / ln
    x = ln * (1.0 / R_MAX)

    li = lax.broadcasted_iota(jnp.int32, (1, 16), 1)
    nv = jnp.where(li < NUM_BESSEL, (li + 1).astype(jnp.float32) * np.pi, 0.0)
    x2 = x * x
    x4 = x2 * x2
    x5 = x4 * x
    env = 1.0 - 21.0 * x5 + 35.0 * x5 * x - 15.0 * x5 * x2
    env = jnp.where(x < 1.0, env, 0.0)
    ef16 = (PREF * env / ln) * jnp.sin(nv * x)

    hid = jnp.maximum(_mm(ef16, f1_ref[...]) + b1_ref[...], 0.0)
    w = _mm(hid, f2_ref[...]) + b2_ref[...]

    dtile = _mm(u16, P3n)
    y1t = SQ3 * dtile

    if first:
        a0 = 1.0 / np.sqrt(EMB)
        a1 = np.sqrt(3.0) / np.sqrt(EMB)
        h0 = hd
        s = a0 * _tp16(h0, w[:, 0:256], R, S)
        g = a0 * _tp16(h0, w[:, 256:512], R, S)
        t = _tp16(h0, w[:, 512:768], R, S)
        v48 = (a1 / SQ3) * (_mm(t, R2) * y1t)
    else:
        a0 = 1.0 / np.sqrt(2 * EMB)
        a1 = np.sqrt(3.0) / np.sqrt(2 * EMB)
        s_in = hd[:, :16]
        v_in = hd[:, 16:64]
        dot = _mm(v_in * dtile, S2)
        s = a0 * (_tp16(s_in, w[:, 0:256], R, S) + _tp16(dot, w[:, 256:512], R, S))
        g = a0 * (_tp16(s_in, w[:, 512:768], R, S) + _tp16(dot, w[:, 768:1024], R, S))
        t4 = _tp16(s_in, w[:, 1024:1280], R, S)
        v1 = _mm(t4, R2) * y1t
        lk = lax.broadcasted_iota(jnp.int32, (3 * EMB, EMB), 0)
        l2 = lax.broadcasted_iota(jnp.int32, (EMB, 3 * EMB), 1)
        v2 = jnp.zeros_like(v1)
        for kk in range(3):
            Skk = S2 * (lk % 3 == kk).astype(jnp.float32)
            Rkk = R2 * (l2 % 3 == kk).astype(jnp.float32)
            v2 = v2 + _mm(_tp16(_mm(v_in, Skk), w[:, 1280:1536], R, S), Rkk)
        v48 = (a1 / SQ3) * (v1 + v2)

    T = s.shape[0]
    ri = pl.program_id(0) * T + lax.broadcasted_iota(jnp.int32, (T, 1), 0)
    msk = (ri < N_EDGES).astype(jnp.float32)
    out_ref[...] = jnp.concatenate([s, g, v48], axis=1) * msk


def _edge_call(first, possrc, tbl_dst, f1, b1, f2, b2):
    T = 256
    E = possrc.shape[0]
    Dt = tbl_dst.shape[1]
    wn = f2.shape[1]
    return pl.pallas_call(
        functools.partial(_edge_body, first),
        grid=(E // T,),
        in_specs=[
            pl.BlockSpec((T, 16), lambda i: (i, 0)),
            pl.BlockSpec((T, Dt), lambda i: (i, 0)),
            pl.BlockSpec((16, MLP_DIM), lambda i: (0, 0)),
            pl.BlockSpec((1, MLP_DIM), lambda i: (0, 0)),
            pl.BlockSpec((MLP_DIM, wn), lambda i: (0, 0)),
            pl.BlockSpec((1, wn), lambda i: (0, 0)),
        ],
        out_specs=pl.BlockSpec((T, 5 * EMB), lambda i: (i, 0)),
        out_shape=jax.ShapeDtypeStruct((E, 5 * EMB), jnp.float32),
    )(possrc, tbl_dst, f1, b1, f2, b2)


def _node_body(a_ref, b_ref, res_ref, out_ref):
    agg = a_ref[...] + b_ref[...]
    sc = agg[:, :EMB]
    s = sc * jax.nn.sigmoid(sc)
    gs = jax.nn.sigmoid(agg[:, EMB:2 * EMB])
    l2 = lax.broadcasted_iota(jnp.int32, (EMB, 3 * EMB), 1)
    r2 = lax.broadcasted_iota(jnp.int32, (EMB, 3 * EMB), 0)
    Rg = (l2 // 3 == r2).astype(jnp.float32)
    v48 = agg[:, 2 * EMB:] * _mm(gs, Rg)
    out_ref[...] = jnp.concatenate([s, v48], axis=1) + res_ref[...]


def _node_call(agg_a, agg_b, res):
    T = 400
    N = agg_a.shape[0]
    return pl.pallas_call(
        _node_body,
        grid=(N // T,),
        in_specs=[
            pl.BlockSpec((T, 5 * EMB), lambda i: (i, 0)),
            pl.BlockSpec((T, 5 * EMB), lambda i: (i, 0)),
            pl.BlockSpec((T, 4 * EMB), lambda i: (i, 0)),
        ],
        out_specs=pl.BlockSpec((T, 4 * EMB), lambda i: (i, 0)),
        out_shape=jax.ShapeDtypeStruct((N, 4 * EMB), jnp.float32),
    )(agg_a, agg_b, res)


def _readout_body(h_ref, batch_ref, w1_ref, b1_ref, w2_ref, b2_ref, out_ref):
    b = batch_ref[...]
    g = lax.broadcasted_iota(jnp.int32, (N_GRAPHS, 1), 0)
    onehot = (b == g).astype(jnp.float32)
    sums = _mm(onehot, h_ref[...][:, :EMB])
    hid = jnp.maximum(_mm(sums, w1_ref[...]) + b1_ref[...], 0.0)
    out_ref[...] = _mm(hid, w2_ref[...]) + b2_ref[...]


def _readout_call(h, batch2d, w1, b1, w2, b2):
    N = h.shape[0]
    return pl.pallas_call(
        _readout_body,
        in_specs=[
            pl.BlockSpec((N, 4 * EMB), lambda: (0, 0)),
            pl.BlockSpec((1, N), lambda: (0, 0)),
            pl.BlockSpec((EMB, EMB), lambda: (0, 0)),
            pl.BlockSpec((1, EMB), lambda: (0, 0)),
            pl.BlockSpec((EMB, 1), lambda: (0, 0)),
            pl.BlockSpec((1, 1), lambda: (0, 0)),
        ],
        out_specs=pl.BlockSpec((N_GRAPHS, 1), lambda: (0, 0)),
        out_shape=jax.ShapeDtypeStruct((N_GRAPHS, 1), jnp.float32),
    )(h, batch2d, w1, b1, w2, b2)


def kernel(atoms, pos, edge_index, batch, emb, fc1_w0, fc1_b0, fc2_w0, fc2_b0,
           fc1_w1, fc1_b1, fc2_w1, fc2_b1, pred_w1, pred_b1, pred_w2, pred_b2):
    f32 = jnp.float32
    src = edge_index[0].astype(jnp.int32)
    dst = edge_index[1].astype(jnp.int32)
    atoms_i = atoms.astype(jnp.int32)

    pos16 = jnp.pad(pos.astype(f32), ((0, 0), (0, 13)))
    zeros80 = jnp.zeros((N_NODES, 5 * EMB), f32)

    EP = NW * 40 * 128

    def idx3(i, iters, chunk):
        B = NW * iters * chunk
        return jnp.pad(i, (0, B - i.shape[0])).reshape(NW, iters, chunk)

    src3 = idx3(src, 40, 128)
    dst3 = idx3(dst, 40, 128)
    atoms3 = idx3(atoms_i, 5, 64)

    h0 = _sc_gather(emb.astype(f32), atoms3)[:N_NODES]
    possrc = _sc_gather(pos16, src3)

    f1_0 = jnp.pad(fc1_w0.astype(f32), ((0, 8), (0, 0)))
    f1_1 = jnp.pad(fc1_w1.astype(f32), ((0, 8), (0, 0)))
    b1_0 = fc1_b0.reshape(1, -1).astype(f32)
    b1_1 = fc1_b1.reshape(1, -1).astype(f32)
    b2_0 = fc2_b0.reshape(1, -1).astype(f32)
    b2_1 = fc2_b1.reshape(1, -1).astype(f32)

    tbl0 = jnp.concatenate([pos16, h0], axis=1)
    td0 = _sc_gather(tbl0, dst3)
    msg0 = _edge_call(True, possrc, td0, f1_0, b1_0, fc2_w0.astype(f32), b2_0)
    agg0 = _sc_scatter_add(msg0, src3, zeros80)
    res0 = jnp.pad(h0, ((0, 0), (0, 3 * EMB)))
    h1 = _node_call(agg0[0], agg0[1], res0)

    tbl1 = jnp.concatenate([pos16, h1], axis=1)
    td1 = _sc_gather(tbl1, dst3)
    msg1 = _edge_call(False, possrc, td1, f1_1, b1_1, fc2_w1.astype(f32), b2_1)
    agg1 = _sc_scatter_add(msg1, src3, zeros80)
    h2 = _node_call(agg1[0], agg1[1], h1)

    batch2d = batch.astype(jnp.int32).reshape(1, N_NODES)
    return _readout_call(h2, batch2d,
                         pred_w1.astype(f32), pred_b1.reshape(1, -1).astype(f32),
                         pred_w2.astype(f32), pred_b2.reshape(1, -1).astype(f32))
